# Initial kernel scaffold; baseline (speedup 1.0000x reference)
import jax
import jax.numpy as jnp
from jax.experimental import pallas as pl


def kernel(x, hs_init, params, edge_index, gate, forward_level):
    raise NotImplementedError("write your pallas kernel here")


# speedup vs baseline: 6.9313x; 6.9313x over previous
import functools

import jax
import jax.numpy as jnp
from jax import lax
from jax.experimental import pallas as pl
from jax.experimental.pallas import tpu as pltpu
from jax.experimental.pallas import tpu_sc as plsc

N = 10000
E = 160000
H = 128
MM = 128
NB = 1000
CHUNK = 128
NCHUNK = E // CHUNK

_f32 = jnp.float32


def _dot(a, b):
    return jnp.dot(a, b, preferred_element_type=_f32)


def _mlp_chain(f, w1, b1, w2, b2, w3, b3):
    h = jnp.maximum(_dot(f, w1) + b1, 0.0)
    h = jnp.maximum(_dot(h, w2) + b2, 0.0)
    return _dot(h, w3) + b3


def _mlp_one_body(f_ref, w1, b1, w2, b2, w3, b3, o_ref):
    o_ref[...] = _mlp_chain(f_ref[...], w1[...], b1[...], w2[...], b2[...],
                            w3[...], b3[...])


def _mlp_pair_body(fa_ref, fb_ref,
                   aw1, ab1, aw2, ab2, aw3, ab3,
                   bw1, bb1, bw2, bb2, bw3, bb3,
                   oa_ref, ob_ref):
    oa_ref[...] = _mlp_chain(fa_ref[...], aw1[...], ab1[...], aw2[...],
                             ab2[...], aw3[...], ab3[...])
    ob_ref[...] = _mlp_chain(fb_ref[...], bw1[...], bb1[...], bw2[...],
                             bb2[...], bw3[...], bb3[...])


def _row_spec(din):
    return pl.BlockSpec((NB, din), lambda i: (i, 0))


def _full_spec(shape):
    return pl.BlockSpec(shape, lambda i: tuple(0 for _ in shape))


def _mlp_weights(p, name):
    return (p[name + '_W1'], p[name + '_b1'].reshape(1, MM),
            p[name + '_W2'], p[name + '_b2'].reshape(1, MM),
            p[name + '_W3'], p[name + '_b3'].reshape(1, H))


def _mlp_one(feat, p, name):
    din = feat.shape[1]
    ws = _mlp_weights(p, name)
    wspecs = [_full_spec(w.shape) for w in ws]
    return pl.pallas_call(
        _mlp_one_body,
        grid=(N // NB,),
        in_specs=[_row_spec(din)] + wspecs,
        out_specs=_row_spec(H),
        out_shape=jax.ShapeDtypeStruct((N, H), _f32),
    )(feat, *ws)


def _mlp_pair(fa, fb, p, na, nb_):
    wsa = _mlp_weights(p, na)
    wsb = _mlp_weights(p, nb_)
    wspecs = [_full_spec(w.shape) for w in wsa + wsb]
    return pl.pallas_call(
        _mlp_pair_body,
        grid=(N // NB,),
        in_specs=[_row_spec(H), _row_spec(H)] + wspecs,
        out_specs=[_row_spec(H), _row_spec(H)],
        out_shape=[jax.ShapeDtypeStruct((N, H), _f32)] * 2,
    )(fa, fb, *wsa, *wsb)


def _gru_math(msg, h, wih_t, bih, whh_t, bhh):
    gi = _dot(msg, wih_t) + bih
    gh = _dot(h, whh_t) + bhh
    r = jax.nn.sigmoid(gi[:, :H] + gh[:, :H])
    z = jax.nn.sigmoid(gi[:, H:2 * H] + gh[:, H:2 * H])
    n = jnp.tanh(gi[:, 2 * H:] + r * gh[:, 2 * H:])
    return (1.0 - z) * n + z * h


def _gru_pair_body(ms0, ms1, mf0, mf1, hs_ref, hf_ref, m_ref,
                   wih_s, bih_s, whh_s, bhh_s,
                   wih_f, bih_f, whh_f, bhh_f,
                   hs_o, hf_o):
    mask = m_ref[...]
    hs = hs_ref[...]
    hf = hf_ref[...]
    new_s = _gru_math(ms0[...] + ms1[...], hs, wih_s[...], bih_s[...],
                      whh_s[...], bhh_s[...])
    new_f = _gru_math(mf0[...] + mf1[...], hf, wih_f[...], bih_f[...],
                      whh_f[...], bhh_f[...])
    hs_o[...] = mask * new_s + (1.0 - mask) * hs
    hf_o[...] = mask * new_f + (1.0 - mask) * hf


def _gru_weights(p, name):
    return (p[name + '_Wih'].T, p[name + '_bih'].reshape(1, 3 * H),
            p[name + '_Whh'].T, p[name + '_bhh'].reshape(1, 3 * H))


def _gru_pair(ms, mf, hs, hf, mask, p, ns_name, nf_name):
    ws = _gru_weights(p, ns_name) + _gru_weights(p, nf_name)
    wspecs = [_full_spec(w.shape) for w in ws]
    nblk = N // NB
    lo = pl.BlockSpec((NB, H), lambda i: (i, 0))
    hi = pl.BlockSpec((NB, H), lambda i: (i + nblk, 0))
    return pl.pallas_call(
        _gru_pair_body,
        grid=(nblk,),
        in_specs=[lo, hi, lo, hi, _row_spec(H), _row_spec(H),
                  pl.BlockSpec((NB, 1), lambda i: (i, 0))] + wspecs,
        out_specs=[_row_spec(H), _row_spec(H)],
        out_shape=[jax.ShapeDtypeStruct((N, H), _f32)] * 2,
    )(ms, ms, mf, mf, hs, hf, mask, *ws)


def _readout_body(hf_ref, w1, b1, w2, b2, w3, b3, o_ref):
    h = jnp.maximum(_dot(hf_ref[...], w1[...]) + b1[...], 0.0)
    h = jnp.maximum(_dot(h, w2[...]) + b2[...], 0.0)
    o_ref[...] = _dot(h, w3[...]) + b3[...]


def _readout(hf, p):
    scale = 1.0 / jnp.sqrt(jnp.float32(1.0 + 1e-5))
    w1 = p['Wp1'] * (scale * p['g1'])[None, :]
    b1 = (p['bp1'] * scale * p['g1'] + p['be1']).reshape(1, MM)
    w2 = p['Wp2'] * (scale * p['g2'])[None, :]
    b2 = (p['bp2'] * scale * p['g2'] + p['be2']).reshape(1, MM)
    ws = (w1, b1, w2, b2, p['Wp3'], p['bp3'].reshape(1, 1))
    wspecs = [_full_spec(w.shape) for w in ws]
    return pl.pallas_call(
        _readout_body,
        grid=(N // NB,),
        in_specs=[_row_spec(H)] + wspecs,
        out_specs=pl.BlockSpec((NB, 1), lambda i: (i, 0)),
        out_shape=jax.ShapeDtypeStruct((N, 1), _f32),
    )(hf, *ws)


@functools.cache
def _make_segsum():
    info = plsc.get_sparse_core_info()
    nc, ns = info.num_cores, info.num_subcores
    nw = nc * ns
    iters = (NCHUNK + nw - 1) // nw
    rows_per_tile = (N // (8 * ns)) * 8
    tail_base = rows_per_tile * ns
    tail_rows = N - tail_base
    zcopies = (rows_per_tile + CHUNK - 1) // CHUNK
    mesh = plsc.VectorSubcoreMesh(core_axis_name="c", subcore_axis_name="s")

    @functools.partial(
        pl.kernel, mesh=mesh,
        out_type=jax.ShapeDtypeStruct((nc * N, H), _f32),
        scratch_types=[
            pltpu.VMEM((CHUNK,), jnp.int32),
            pltpu.VMEM((CHUNK,), jnp.int32),
            pltpu.VMEM((CHUNK, H), _f32),
            pltpu.VMEM_SHARED((N, H), _f32),
            pltpu.SemaphoreType.DMA,
        ])
    def segsum(u_hbm, src_hbm, dst_hbm, out_hbm, idx_s, idx_d, rows, acc, sem):
        cid = lax.axis_index("c")
        sid = lax.axis_index("s")
        wid = sid * nc + cid

        def zrow(j, _):
            rows[j // (H // 16), pl.ds((j % (H // 16)) * 16, 16)] = (
                jnp.zeros((16,), _f32))
            return 0
        lax.fori_loop(0, CHUNK * (H // 16), zrow, 0)
        tbase = sid * rows_per_tile
        for k in range(zcopies):
            nrows = min(CHUNK, rows_per_tile - k * CHUNK)
            pltpu.sync_copy(rows.at[pl.ds(0, nrows)],
                            acc.at[pl.ds(tbase + k * CHUNK, nrows)])

        @pl.when(sid == 0)
        def _():
            pltpu.sync_copy(rows.at[pl.ds(0, tail_rows)],
                            acc.at[pl.ds(tail_base, tail_rows)])
        plsc.subcore_barrier()

        def ebody(i, _):
            chunk = i * nw + wid

            @pl.when(chunk < NCHUNK)
            def _():
                base = chunk * CHUNK
                pltpu.sync_copy(src_hbm.at[pl.ds(base, CHUNK)], idx_s)
                pltpu.sync_copy(dst_hbm.at[pl.ds(base, CHUNK)], idx_d)
                pltpu.async_copy(u_hbm.at[idx_s], rows, sem).wait()
                pltpu.sync_copy(rows, acc.at[idx_d], add=True)
            return 0
        lax.fori_loop(0, iters, ebody, 0)
        plsc.subcore_barrier()

        pltpu.sync_copy(acc.at[pl.ds(tbase, rows_per_tile)],
                        out_hbm.at[pl.ds(cid * N + tbase, rows_per_tile)])

        @pl.when(sid == 0)
        def _():
            pltpu.sync_copy(acc.at[pl.ds(tail_base, tail_rows)],
                            out_hbm.at[pl.ds(cid * N + tail_base, tail_rows)])

    return segsum


def _segsum_call(u, src, dst):
    return _make_segsum()(u, src, dst)


def kernel(x, hs_init, params, edge_index, gate, forward_level):
    p = params
    src = edge_index[0]
    dst = edge_index[1]
    hs = hs_init
    hf = jnp.broadcast_to(p['We'] + p['be'], (N, H))
    node_state = jnp.concatenate([hs, hf], axis=-1)
    and_m = gate == 1
    not_m = gate == 2

    u_af = _mlp_one(node_state, p, 'af')

    for level in (1, 2, 3):
        lm = forward_level == level
        sa = (lm & and_m).astype(_f32).reshape(N, 1)
        sn = (lm & not_m).astype(_f32).reshape(N, 1)

        u_as = _mlp_one(hs, p, 'as')
        ms = _segsum_call(u_as, src, dst)
        mf = _segsum_call(u_af, src, dst)
        hs, hf = _gru_pair(ms, mf, hs, hf, sa, p, 'gas', 'gaf')

        u_ns, u_nf = _mlp_pair(hs, hf, p, 'ns', 'nf')
        ms = _segsum_call(u_ns, src, dst)
        mf = _segsum_call(u_nf, src, dst)
        hs, hf = _gru_pair(ms, mf, hs, hf, sn, p, 'gns', 'gnf')

    prob = _readout(hf, p)
    return hs, hf, prob, jnp.float32(0.0)


# speedup vs baseline: 9.7681x; 1.4093x over previous
import functools

import jax
import jax.numpy as jnp
from jax import lax
from jax.experimental import pallas as pl
from jax.experimental.pallas import tpu as pltpu
from jax.experimental.pallas import tpu_sc as plsc

N = 10000
E = 160000
H = 128
MM = 128
NB = 1000
CHUNK = 128
NCHUNK = E // CHUNK
SELW = 512

_f32 = jnp.float32


def _pack_bits(sel_bool):
    bits = jnp.zeros((SELW * 32,), jnp.uint32).at[:N].set(
        sel_bool.astype(jnp.uint32))
    words = (bits.reshape(SELW, 32)
             << jnp.arange(32, dtype=jnp.uint32)[None, :]).sum(
        axis=1, dtype=jnp.uint32)
    return jax.lax.bitcast_convert_type(words, jnp.int32)


def _dot(a, b):
    return jnp.dot(a, b, preferred_element_type=_f32)


def _mlp_chain(f, w1, b1, w2, b2, w3, b3):
    h = jnp.maximum(_dot(f, w1) + b1, 0.0)
    h = jnp.maximum(_dot(h, w2) + b2, 0.0)
    return _dot(h, w3) + b3


def _mlp_one_body(f_ref, w1, b1, w2, b2, w3, b3, o_ref):
    o_ref[...] = _mlp_chain(f_ref[...], w1[...], b1[...], w2[...], b2[...],
                            w3[...], b3[...])


def _mlp_pair_body(fa_ref, fb_ref,
                   aw1, ab1, aw2, ab2, aw3, ab3,
                   bw1, bb1, bw2, bb2, bw3, bb3,
                   oa_ref, ob_ref):
    oa_ref[...] = _mlp_chain(fa_ref[...], aw1[...], ab1[...], aw2[...],
                             ab2[...], aw3[...], ab3[...])
    ob_ref[...] = _mlp_chain(fb_ref[...], bw1[...], bb1[...], bw2[...],
                             bb2[...], bw3[...], bb3[...])


def _row_spec(din):
    return pl.BlockSpec((NB, din), lambda i: (i, 0))


def _full_spec(shape):
    return pl.BlockSpec(shape, lambda i: tuple(0 for _ in shape))


def _mlp_weights(p, name):
    return (p[name + '_W1'], p[name + '_b1'].reshape(1, MM),
            p[name + '_W2'], p[name + '_b2'].reshape(1, MM),
            p[name + '_W3'], p[name + '_b3'].reshape(1, H))


def _mlp_one(feat, p, name):
    din = feat.shape[1]
    ws = _mlp_weights(p, name)
    wspecs = [_full_spec(w.shape) for w in ws]
    return pl.pallas_call(
        _mlp_one_body,
        grid=(N // NB,),
        in_specs=[_row_spec(din)] + wspecs,
        out_specs=_row_spec(H),
        out_shape=jax.ShapeDtypeStruct((N, H), _f32),
    )(feat, *ws)


def _mlp_pair(fa, fb, p, na, nb_):
    wsa = _mlp_weights(p, na)
    wsb = _mlp_weights(p, nb_)
    wspecs = [_full_spec(w.shape) for w in wsa + wsb]
    return pl.pallas_call(
        _mlp_pair_body,
        grid=(N // NB,),
        in_specs=[_row_spec(H), _row_spec(H)] + wspecs,
        out_specs=[_row_spec(H), _row_spec(H)],
        out_shape=[jax.ShapeDtypeStruct((N, H), _f32)] * 2,
    )(fa, fb, *wsa, *wsb)


def _gru_math(msg, h, wih_t, bih, whh_t, bhh):
    gi = _dot(msg, wih_t) + bih
    gh = _dot(h, whh_t) + bhh
    r = jax.nn.sigmoid(gi[:, :H] + gh[:, :H])
    z = jax.nn.sigmoid(gi[:, H:2 * H] + gh[:, H:2 * H])
    n = jnp.tanh(gi[:, 2 * H:] + r * gh[:, 2 * H:])
    return (1.0 - z) * n + z * h


def _gru_pair_body(ms0, ms1, mf0, mf1, hs_ref, hf_ref, m_ref,
                   wih_s, bih_s, whh_s, bhh_s,
                   wih_f, bih_f, whh_f, bhh_f,
                   hs_o, hf_o):
    mask = m_ref[...]
    hs = hs_ref[...]
    hf = hf_ref[...]
    new_s = _gru_math(ms0[...] + ms1[...], hs, wih_s[...], bih_s[...],
                      whh_s[...], bhh_s[...])
    new_f = _gru_math(mf0[...] + mf1[...], hf, wih_f[...], bih_f[...],
                      whh_f[...], bhh_f[...])
    hs_o[...] = mask * new_s + (1.0 - mask) * hs
    hf_o[...] = mask * new_f + (1.0 - mask) * hf


def _gru_weights(p, name):
    return (p[name + '_Wih'].T, p[name + '_bih'].reshape(1, 3 * H),
            p[name + '_Whh'].T, p[name + '_bhh'].reshape(1, 3 * H))


def _gru_pair(ms, mf, hs, hf, mask, p, ns_name, nf_name):
    ws = _gru_weights(p, ns_name) + _gru_weights(p, nf_name)
    wspecs = [_full_spec(w.shape) for w in ws]
    nblk = N // NB
    lo = pl.BlockSpec((NB, H), lambda i: (i, 0))
    hi = pl.BlockSpec((NB, H), lambda i: (i + nblk, 0))
    return pl.pallas_call(
        _gru_pair_body,
        grid=(nblk,),
        in_specs=[lo, hi, lo, hi, _row_spec(H), _row_spec(H),
                  pl.BlockSpec((NB, 1), lambda i: (i, 0))] + wspecs,
        out_specs=[_row_spec(H), _row_spec(H)],
        out_shape=[jax.ShapeDtypeStruct((N, H), _f32)] * 2,
    )(ms, ms, mf, mf, hs, hf, mask, *ws)


def _readout_body(hf_ref, w1, b1, w2, b2, w3, b3, o_ref):
    h = jnp.maximum(_dot(hf_ref[...], w1[...]) + b1[...], 0.0)
    h = jnp.maximum(_dot(h, w2[...]) + b2[...], 0.0)
    o_ref[...] = _dot(h, w3[...]) + b3[...]


def _readout(hf, p):
    scale = 1.0 / jnp.sqrt(jnp.float32(1.0 + 1e-5))
    w1 = p['Wp1'] * (scale * p['g1'])[None, :]
    b1 = (p['bp1'] * scale * p['g1'] + p['be1']).reshape(1, MM)
    w2 = p['Wp2'] * (scale * p['g2'])[None, :]
    b2 = (p['bp2'] * scale * p['g2'] + p['be2']).reshape(1, MM)
    ws = (w1, b1, w2, b2, p['Wp3'], p['bp3'].reshape(1, 1))
    wspecs = [_full_spec(w.shape) for w in ws]
    return pl.pallas_call(
        _readout_body,
        grid=(N // NB,),
        in_specs=[_row_spec(H)] + wspecs,
        out_specs=pl.BlockSpec((NB, 1), lambda i: (i, 0)),
        out_shape=jax.ShapeDtypeStruct((N, 1), _f32),
    )(hf, *ws)


@functools.cache
def _make_segsum_pair():
    info = plsc.get_sparse_core_info()
    nc, ns = info.num_cores, info.num_subcores
    nw = nc * ns
    ept = E // nw
    stg = 2048
    gch = 64
    maxch = (ept + gch - 1) // gch + 1
    rows_per_tile = (N // (8 * ns)) * 8
    tail_base = rows_per_tile * ns
    tail_rows = N - tail_base
    zcopies = (rows_per_tile + gch - 1) // gch
    mesh = plsc.VectorSubcoreMesh(core_axis_name="c", subcore_axis_name="s")

    @functools.partial(
        pl.kernel, mesh=mesh,
        out_type=[jax.ShapeDtypeStruct((nc * N, H), _f32)] * 2,
        compiler_params=pltpu.CompilerParams(needs_layout_passes=False),
        scratch_types=[
            pltpu.VMEM((stg,), jnp.int32),
            pltpu.VMEM((stg,), jnp.int32),
            pltpu.VMEM((SELW,), jnp.int32),
            pltpu.VMEM((2 * gch,), jnp.int32),
            pltpu.VMEM((2 * gch,), jnp.int32),
            pltpu.VMEM((maxch, gch), jnp.int32),
            pltpu.VMEM((maxch, gch), jnp.int32),
            pltpu.VMEM((gch, H), _f32),
            pltpu.VMEM((gch, H), _f32),
            pltpu.VMEM_SHARED((N + 8, H), _f32),
            pltpu.SemaphoreType.DMA,
            pltpu.SemaphoreType.DMA,
        ])
    def segsum2(u1_hbm, u2_hbm, src_hbm, dst_hbm, sel_hbm, o1_hbm, o2_hbm,
                es, ed, sel_v, pend_s, pend_d, cs2, cd2,
                rows_a, rows_b, acc, sem_a, sem_b):
        cid = lax.axis_index("c")
        sid = lax.axis_index("s")
        wid = sid * nc + cid
        tbase = sid * rows_per_tile

        pltpu.sync_copy(sel_hbm, sel_v)

        def zero_rows_a():
            def zrow(j, _):
                rows_a[j // (H // 16), pl.ds((j % (H // 16)) * 16, 16)] = (
                    jnp.zeros((16,), _f32))
                return 0
            lax.fori_loop(0, gch * (H // 16), zrow, 0)

        def zero_acc():
            for k in range(zcopies):
                nrows = min(gch, rows_per_tile - k * gch)
                pltpu.async_copy(rows_a.at[pl.ds(0, nrows)],
                                 acc.at[pl.ds(tbase + k * gch, nrows)], sem_a)

            @pl.when(sid == 0)
            def _():
                pltpu.async_copy(rows_a.at[pl.ds(0, tail_rows)],
                                 acc.at[pl.ds(tail_base, tail_rows)], sem_a)
            for k in range(zcopies):
                nrows = min(gch, rows_per_tile - k * gch)
                pltpu.make_async_copy(
                    rows_a.at[pl.ds(0, nrows)],
                    acc.at[pl.ds(tbase + k * gch, nrows)], sem_a).wait()

            @pl.when(sid == 0)
            def _():
                pltpu.make_async_copy(
                    rows_a.at[pl.ds(0, tail_rows)],
                    acc.at[pl.ds(tail_base, tail_rows)], sem_a).wait()

        def flush(carry):
            pcnt, crow = carry
            for k in range(gch // 16):
                sl = pl.ds(k * 16, 16)
                cs2[crow, sl] = pend_s[sl]
                cd2[crow, sl] = pend_d[sl]
                pend_s[sl] = pend_s[pl.ds(gch + k * 16, 16)]
                pend_d[sl] = pend_d[pl.ds(gch + k * 16, 16)]
            return pcnt - gch, crow + 1

        lane = lax.iota(jnp.int32, 16)

        def make_group_body(half_len):
            def group_body(g, carry):
                pcnt, crow = carry
                srcv = es[pl.ds(g * 16, 16)]
                dstv = ed[pl.ds(g * 16, 16)]
                w = plsc.load_gather(sel_v, [lax.shift_right_logical(dstv, 5)])
                m = (lax.shift_right_logical(w, dstv & 31) & 1) > 0
                if half_len % 16 != 0:
                    m = m & (g * 16 + lane < half_len)
                plsc.store_compressed(pend_s.at[pl.ds(pcnt, 16)], srcv,
                                      mask=m)
                plsc.store_compressed(pend_d.at[pl.ds(pcnt, 16)], dstv,
                                      mask=m)
                pcnt = pcnt + jnp.sum(m.astype(jnp.int32))
                return lax.cond(pcnt >= gch, flush, lambda c: c, (pcnt, crow))
            return group_body

        carry = (jnp.int32(0), jnp.int32(0))
        hbase = 0
        while hbase < ept:
            hlen = min(stg, ept - hbase)
            pltpu.sync_copy(src_hbm.at[pl.ds(wid * ept + hbase, hlen)],
                            es.at[pl.ds(0, hlen)])
            pltpu.sync_copy(dst_hbm.at[pl.ds(wid * ept + hbase, hlen)],
                            ed.at[pl.ds(0, hlen)])
            carry = lax.fori_loop(0, (hlen + 15) // 16,
                                  make_group_body(hlen), carry)
            hbase += hlen
        pcnt, crow = carry
        dummy_s = jnp.zeros((16,), jnp.int32)
        dummy_d = jnp.full((16,), N, jnp.int32)
        for k in range(gch // 16):
            pend_s[pl.ds(pcnt + k * 16, 16)] = dummy_s
            pend_d[pl.ds(pcnt + k * 16, 16)] = dummy_d
        _, crow = flush((pcnt, crow))
        nch = crow

        def sweep(u_hbm):
            @pl.when(nch > 0)
            def _():
                pltpu.async_copy(u_hbm.at[cs2.at[0]], rows_a, sem_a)

            def pair_body(j2, _):
                c0 = j2 * 2
                c1 = c0 + 1

                @pl.when(c0 < nch)
                def _():
                    pltpu.make_async_copy(
                        u_hbm.at[cs2.at[c0]], rows_a, sem_a).wait()

                    @pl.when(c1 < nch)
                    def _():
                        pltpu.async_copy(u_hbm.at[cs2.at[c1]], rows_b, sem_b)
                    pltpu.sync_copy(rows_a, acc.at[cd2.at[c0]], add=True)

                @pl.when(c1 < nch)
                def _():
                    pltpu.make_async_copy(
                        u_hbm.at[cs2.at[c1]], rows_b, sem_b).wait()

                    @pl.when(c0 + 2 < nch)
                    def _():
                        pltpu.async_copy(u_hbm.at[cs2.at[c0 + 2]], rows_a,
                                         sem_a)
                    pltpu.sync_copy(rows_b, acc.at[cd2.at[c1]], add=True)
                return 0
            lax.fori_loop(0, (nch + 1) // 2, pair_body, 0)

        def dump(o_hbm):
            pltpu.sync_copy(acc.at[pl.ds(tbase, rows_per_tile)],
                            o_hbm.at[pl.ds(cid * N + tbase, rows_per_tile)])

            @pl.when(sid == 0)
            def _():
                pltpu.sync_copy(acc.at[pl.ds(tail_base, tail_rows)],
                                o_hbm.at[pl.ds(cid * N + tail_base,
                                               tail_rows)])

        zero_rows_a()
        zero_acc()
        plsc.subcore_barrier()
        sweep(u1_hbm)
        plsc.subcore_barrier()
        dump(o1_hbm)
        zero_rows_a()
        zero_acc()
        plsc.subcore_barrier()
        sweep(u2_hbm)
        plsc.subcore_barrier()
        dump(o2_hbm)

    return segsum2


def _segsum_pair_call(u1, u2, src, dst, sel):
    return _make_segsum_pair()(u1, u2, src, dst, sel)


def kernel(x, hs_init, params, edge_index, gate, forward_level):
    p = params
    src = edge_index[0]
    dst = edge_index[1]
    hs = hs_init
    hf = jnp.broadcast_to(p['We'] + p['be'], (N, H))
    node_state = jnp.concatenate([hs, hf], axis=-1)
    and_m = gate == 1
    not_m = gate == 2

    u_af = _mlp_one(node_state, p, 'af')

    for level in (1, 2, 3):
        lm = forward_level == level
        sa_b = lm & and_m
        sn_b = lm & not_m
        sa = sa_b.astype(_f32).reshape(N, 1)
        sn = sn_b.astype(_f32).reshape(N, 1)
        sa_i = _pack_bits(sa_b)
        sn_i = _pack_bits(sn_b)

        u_as = _mlp_one(hs, p, 'as')
        ms, mf = _segsum_pair_call(u_as, u_af, src, dst, sa_i)
        hs, hf = _gru_pair(ms, mf, hs, hf, sa, p, 'gas', 'gaf')

        u_ns, u_nf = _mlp_pair(hs, hf, p, 'ns', 'nf')
        ms, mf = _segsum_pair_call(u_ns, u_nf, src, dst, sn_i)
        hs, hf = _gru_pair(ms, mf, hs, hf, sn, p, 'gns', 'gnf')

    prob = _readout(hf, p)
    return hs, hf, prob, jnp.float32(0.0)


# speedup vs baseline: 10.5032x; 1.0753x over previous
import functools

import jax
import jax.numpy as jnp
from jax import lax
from jax.experimental import pallas as pl
from jax.experimental.pallas import tpu as pltpu
from jax.experimental.pallas import tpu_sc as plsc

N = 10000
E = 160000
H = 128
MM = 128
NB = 1000
CHUNK = 128
NCHUNK = E // CHUNK
SELW = 512

_f32 = jnp.float32


def _pack_bits(sel_bool):
    bits = jnp.zeros((SELW * 32,), jnp.uint32).at[:N].set(
        sel_bool.astype(jnp.uint32))
    words = (bits.reshape(SELW, 32)
             << jnp.arange(32, dtype=jnp.uint32)[None, :]).sum(
        axis=1, dtype=jnp.uint32)
    return jax.lax.bitcast_convert_type(words, jnp.int32)


def _dot(a, b):
    return jnp.dot(a, b, preferred_element_type=_f32)


def _mlp_chain(f, w1, b1, w2, b2, w3, b3):
    h = jnp.maximum(_dot(f, w1) + b1, 0.0)
    h = jnp.maximum(_dot(h, w2) + b2, 0.0)
    return _dot(h, w3) + b3


def _mlp_one_body(f_ref, w1, b1, w2, b2, w3, b3, o_ref):
    o_ref[...] = _mlp_chain(f_ref[...], w1[...], b1[...], w2[...], b2[...],
                            w3[...], b3[...])


def _mlp_pair_body(fa_ref, fb_ref,
                   aw1, ab1, aw2, ab2, aw3, ab3,
                   bw1, bb1, bw2, bb2, bw3, bb3,
                   oa_ref, ob_ref):
    oa_ref[...] = _mlp_chain(fa_ref[...], aw1[...], ab1[...], aw2[...],
                             ab2[...], aw3[...], ab3[...])
    ob_ref[...] = _mlp_chain(fb_ref[...], bw1[...], bb1[...], bw2[...],
                             bb2[...], bw3[...], bb3[...])


def _row_spec(din):
    return pl.BlockSpec((NB, din), lambda i: (i, 0))


def _full_spec(shape):
    return pl.BlockSpec(shape, lambda i: tuple(0 for _ in shape))


def _mlp_weights(p, name):
    return (p[name + '_W1'], p[name + '_b1'].reshape(1, MM),
            p[name + '_W2'], p[name + '_b2'].reshape(1, MM),
            p[name + '_W3'], p[name + '_b3'].reshape(1, H))


def _mlp_one(feat, p, name):
    din = feat.shape[1]
    ws = _mlp_weights(p, name)
    wspecs = [_full_spec(w.shape) for w in ws]
    return pl.pallas_call(
        _mlp_one_body,
        grid=(N // NB,),
        in_specs=[_row_spec(din)] + wspecs,
        out_specs=_row_spec(H),
        out_shape=jax.ShapeDtypeStruct((N, H), _f32),
    )(feat, *ws)


def _mlp_pair(fa, fb, p, na, nb_):
    wsa = _mlp_weights(p, na)
    wsb = _mlp_weights(p, nb_)
    wspecs = [_full_spec(w.shape) for w in wsa + wsb]
    return pl.pallas_call(
        _mlp_pair_body,
        grid=(N // NB,),
        in_specs=[_row_spec(fa.shape[1]), _row_spec(fb.shape[1])] + wspecs,
        out_specs=[_row_spec(H), _row_spec(H)],
        out_shape=[jax.ShapeDtypeStruct((N, H), _f32)] * 2,
    )(fa, fb, *wsa, *wsb)


def _gru_math(msg, h, wih_t, bih, whh_t, bhh):
    gi = _dot(msg, wih_t) + bih
    gh = _dot(h, whh_t) + bhh
    r = jax.nn.sigmoid(gi[:, :H] + gh[:, :H])
    z = jax.nn.sigmoid(gi[:, H:2 * H] + gh[:, H:2 * H])
    n = jnp.tanh(gi[:, 2 * H:] + r * gh[:, 2 * H:])
    return (1.0 - z) * n + z * h


def _gru_pair_math(ms0, ms1, mf0, mf1, hs, hf, m_ref,
                   wih_s, bih_s, whh_s, bhh_s, wih_f, bih_f, whh_f, bhh_f):
    sel = m_ref[...] > 0.5
    new_s = _gru_math(ms0 + ms1, hs, wih_s[...], bih_s[...],
                      whh_s[...], bhh_s[...])
    new_f = _gru_math(mf0 + mf1, hf, wih_f[...], bih_f[...],
                      whh_f[...], bhh_f[...])
    return jnp.where(sel, new_s, hs), jnp.where(sel, new_f, hf)


def _gru_weights(p, name):
    return (p[name + '_Wih'].T, p[name + '_bih'].reshape(1, 3 * H),
            p[name + '_Whh'].T, p[name + '_bhh'].reshape(1, 3 * H))


def _gru_mlp_body(ms0, ms1, mf0, mf1, hs_ref, hf_ref, m_ref,
                  gw1, gw2, gw3, gw4, gw5, gw6, gw7, gw8,
                  aw1, ab1, aw2, ab2, aw3, ab3,
                  bw1, bb1, bw2, bb2, bw3, bb3,
                  hs_o, hf_o, ua_o, ub_o, *, b_input):
    hs_n, hf_n = _gru_pair_math(
        ms0[...], ms1[...], mf0[...], mf1[...], hs_ref[...], hf_ref[...],
        m_ref, gw1, gw2, gw3, gw4, gw5, gw6, gw7, gw8)
    hs_o[...] = hs_n
    hf_o[...] = hf_n
    ua_o[...] = _mlp_chain(hs_n, aw1[...], ab1[...], aw2[...], ab2[...],
                           aw3[...], ab3[...])
    if b_input == 'dup':
        ub_o[...] = ua_o[...]
    else:
        ub_o[...] = _mlp_chain(hf_n, bw1[...], bb1[...], bw2[...], bb2[...],
                               bw3[...], bb3[...])


def _gru_mlp(ms, mf, hs, hf, mask, p, gs_name, gf_name, ma_name, mb_name,
             b_input):
    gws = _gru_weights(p, gs_name) + _gru_weights(p, gf_name)
    wsa = _mlp_weights(p, ma_name)
    wsb = _mlp_weights(p, mb_name)
    ws = gws + wsa + wsb
    wspecs = [_full_spec(w.shape) for w in ws]
    nblk = N // NB
    lo = pl.BlockSpec((NB, H), lambda i: (i, 0))
    hi = pl.BlockSpec((NB, H), lambda i: (i + nblk, 0))
    body = functools.partial(_gru_mlp_body, b_input=b_input)
    return pl.pallas_call(
        body,
        grid=(nblk,),
        in_specs=[lo, hi, lo, hi, _row_spec(H), _row_spec(H),
                  pl.BlockSpec((NB, 1), lambda i: (i, 0))] + wspecs,
        out_specs=[_row_spec(H)] * 4,
        out_shape=[jax.ShapeDtypeStruct((N, H), _f32)] * 4,
    )(ms, ms, mf, mf, hs, hf, mask, *ws)


def _gru_readout_body(ms0, ms1, mf0, mf1, hs_ref, hf_ref, m_ref,
                      gw1, gw2, gw3, gw4, gw5, gw6, gw7, gw8,
                      rw1, rb1, rw2, rb2, rw3, rb3,
                      hs_o, hf_o, pr_o):
    hs_n, hf_n = _gru_pair_math(
        ms0[...], ms1[...], mf0[...], mf1[...], hs_ref[...], hf_ref[...],
        m_ref, gw1, gw2, gw3, gw4, gw5, gw6, gw7, gw8)
    hs_o[...] = hs_n
    hf_o[...] = hf_n
    h = jnp.maximum(_dot(hf_n, rw1[...]) + rb1[...], 0.0)
    h = jnp.maximum(_dot(h, rw2[...]) + rb2[...], 0.0)
    pr_o[...] = _dot(h, rw3[...]) + rb3[...]


def _readout_weights(p):
    scale = 1.0 / jnp.sqrt(jnp.float32(1.0 + 1e-5))
    w1 = p['Wp1'] * (scale * p['g1'])[None, :]
    b1 = (p['bp1'] * scale * p['g1'] + p['be1']).reshape(1, MM)
    w2 = p['Wp2'] * (scale * p['g2'])[None, :]
    b2 = (p['bp2'] * scale * p['g2'] + p['be2']).reshape(1, MM)
    return (w1, b1, w2, b2, p['Wp3'], p['bp3'].reshape(1, 1))


def _gru_readout(ms, mf, hs, hf, mask, p, gs_name, gf_name):
    ws = _gru_weights(p, gs_name) + _gru_weights(p, gf_name)
    ws = ws + _readout_weights(p)
    wspecs = [_full_spec(w.shape) for w in ws]
    nblk = N // NB
    lo = pl.BlockSpec((NB, H), lambda i: (i, 0))
    hi = pl.BlockSpec((NB, H), lambda i: (i + nblk, 0))
    return pl.pallas_call(
        _gru_readout_body,
        grid=(nblk,),
        in_specs=[lo, hi, lo, hi, _row_spec(H), _row_spec(H),
                  pl.BlockSpec((NB, 1), lambda i: (i, 0))] + wspecs,
        out_specs=[_row_spec(H), _row_spec(H),
                   pl.BlockSpec((NB, 1), lambda i: (i, 0))],
        out_shape=[jax.ShapeDtypeStruct((N, H), _f32),
                   jax.ShapeDtypeStruct((N, H), _f32),
                   jax.ShapeDtypeStruct((N, 1), _f32)],
    )(ms, ms, mf, mf, hs, hf, mask, *ws)


@functools.cache
def _make_segsum_pair():
    info = plsc.get_sparse_core_info()
    nc, ns = info.num_cores, info.num_subcores
    nw = nc * ns
    ept = E // nw
    stg = 2048
    gch = 64
    maxch = (ept + gch - 1) // gch + 1
    rows_per_tile = (N // (8 * ns)) * 8
    tail_base = rows_per_tile * ns
    tail_rows = N - tail_base
    zcopies = (rows_per_tile + gch - 1) // gch
    mesh = plsc.VectorSubcoreMesh(core_axis_name="c", subcore_axis_name="s")

    @functools.partial(
        pl.kernel, mesh=mesh,
        out_type=[jax.ShapeDtypeStruct((nc * N, H), _f32)] * 2,
        compiler_params=pltpu.CompilerParams(needs_layout_passes=False),
        scratch_types=[
            pltpu.VMEM((stg,), jnp.int32),
            pltpu.VMEM((stg,), jnp.int32),
            pltpu.VMEM((SELW,), jnp.int32),
            pltpu.VMEM((2 * gch,), jnp.int32),
            pltpu.VMEM((2 * gch,), jnp.int32),
            pltpu.VMEM((maxch, gch), jnp.int32),
            pltpu.VMEM((maxch, gch), jnp.int32),
            pltpu.VMEM((gch, H), _f32),
            pltpu.VMEM((gch, H), _f32),
            pltpu.VMEM_SHARED((N + 8, H), _f32),
            pltpu.SemaphoreType.DMA,
            pltpu.SemaphoreType.DMA,
        ])
    def segsum2(u1_hbm, u2_hbm, src_hbm, dst_hbm, sel_hbm, o1_hbm, o2_hbm,
                es, ed, sel_v, pend_s, pend_d, cs2, cd2,
                rows_a, rows_b, acc, sem_a, sem_b):
        cid = lax.axis_index("c")
        sid = lax.axis_index("s")
        wid = sid * nc + cid
        tbase = sid * rows_per_tile

        pltpu.sync_copy(sel_hbm, sel_v)

        def zero_rows_a():
            def zrow(j, _):
                rows_a[j // (H // 16), pl.ds((j % (H // 16)) * 16, 16)] = (
                    jnp.zeros((16,), _f32))
                return 0
            lax.fori_loop(0, gch * (H // 16), zrow, 0)

        def zero_acc_fire():
            for k in range(zcopies):
                nrows = min(gch, rows_per_tile - k * gch)
                pltpu.async_copy(rows_a.at[pl.ds(0, nrows)],
                                 acc.at[pl.ds(tbase + k * gch, nrows)], sem_a)

            @pl.when(sid == 0)
            def _():
                pltpu.async_copy(rows_a.at[pl.ds(0, tail_rows)],
                                 acc.at[pl.ds(tail_base, tail_rows)], sem_a)

        def zero_acc_drain():
            for k in range(zcopies):
                nrows = min(gch, rows_per_tile - k * gch)
                pltpu.make_async_copy(
                    rows_a.at[pl.ds(0, nrows)],
                    acc.at[pl.ds(tbase + k * gch, nrows)], sem_a).wait()

            @pl.when(sid == 0)
            def _():
                pltpu.make_async_copy(
                    rows_a.at[pl.ds(0, tail_rows)],
                    acc.at[pl.ds(tail_base, tail_rows)], sem_a).wait()

        def flush(carry):
            pcnt, crow = carry
            for k in range(gch // 16):
                sl = pl.ds(k * 16, 16)
                cs2[crow, sl] = pend_s[sl]
                cd2[crow, sl] = pend_d[sl]
                pend_s[sl] = pend_s[pl.ds(gch + k * 16, 16)]
                pend_d[sl] = pend_d[pl.ds(gch + k * 16, 16)]
            return pcnt - gch, crow + 1

        lane = lax.iota(jnp.int32, 16)

        def make_group_body(half_len):
            def group_body(g, carry):
                pcnt, crow = carry
                srcv = es[pl.ds(g * 16, 16)]
                dstv = ed[pl.ds(g * 16, 16)]
                w = plsc.load_gather(sel_v, [lax.shift_right_logical(dstv, 5)])
                m = (lax.shift_right_logical(w, dstv & 31) & 1) > 0
                if half_len % 16 != 0:
                    m = m & (g * 16 + lane < half_len)
                plsc.store_compressed(pend_s.at[pl.ds(pcnt, 16)], srcv,
                                      mask=m)
                plsc.store_compressed(pend_d.at[pl.ds(pcnt, 16)], dstv,
                                      mask=m)
                pcnt = pcnt + jnp.sum(m.astype(jnp.int32))
                return lax.cond(pcnt >= gch, flush, lambda c: c, (pcnt, crow))
            return group_body

        zero_rows_a()
        zero_acc_fire()

        carry = (jnp.int32(0), jnp.int32(0))
        hbase = 0
        while hbase < ept:
            hlen = min(stg, ept - hbase)
            pltpu.sync_copy(src_hbm.at[pl.ds(wid * ept + hbase, hlen)],
                            es.at[pl.ds(0, hlen)])
            pltpu.sync_copy(dst_hbm.at[pl.ds(wid * ept + hbase, hlen)],
                            ed.at[pl.ds(0, hlen)])
            carry = lax.fori_loop(0, (hlen + 15) // 16,
                                  make_group_body(hlen), carry)
            hbase += hlen
        pcnt, crow = carry
        dummy_s = jnp.zeros((16,), jnp.int32)
        dummy_d = jnp.full((16,), N, jnp.int32)
        for k in range(gch // 16):
            pend_s[pl.ds(pcnt + k * 16, 16)] = dummy_s
            pend_d[pl.ds(pcnt + k * 16, 16)] = dummy_d
        _, crow = flush((pcnt, crow))
        nch = crow

        def sweep(u_hbm):
            @pl.when(nch > 0)
            def _():
                pltpu.async_copy(u_hbm.at[cs2.at[0]], rows_a, sem_a)

            def pair_body(j2, _):
                c0 = j2 * 2
                c1 = c0 + 1

                @pl.when(c0 < nch)
                def _():
                    pltpu.make_async_copy(
                        u_hbm.at[cs2.at[c0]], rows_a, sem_a).wait()

                    @pl.when(c1 < nch)
                    def _():
                        pltpu.async_copy(u_hbm.at[cs2.at[c1]], rows_b, sem_b)
                    pltpu.sync_copy(rows_a, acc.at[cd2.at[c0]], add=True)

                @pl.when(c1 < nch)
                def _():
                    pltpu.make_async_copy(
                        u_hbm.at[cs2.at[c1]], rows_b, sem_b).wait()

                    @pl.when(c0 + 2 < nch)
                    def _():
                        pltpu.async_copy(u_hbm.at[cs2.at[c0 + 2]], rows_a,
                                         sem_a)
                    pltpu.sync_copy(rows_b, acc.at[cd2.at[c1]], add=True)
                return 0
            lax.fori_loop(0, (nch + 1) // 2, pair_body, 0)

        def dump(o_hbm):
            pltpu.sync_copy(acc.at[pl.ds(tbase, rows_per_tile)],
                            o_hbm.at[pl.ds(cid * N + tbase, rows_per_tile)])

            @pl.when(sid == 0)
            def _():
                pltpu.sync_copy(acc.at[pl.ds(tail_base, tail_rows)],
                                o_hbm.at[pl.ds(cid * N + tail_base,
                                               tail_rows)])

        zero_acc_drain()
        plsc.subcore_barrier()
        sweep(u1_hbm)
        plsc.subcore_barrier()
        dump(o1_hbm)
        zero_rows_a()
        zero_acc_fire()
        zero_acc_drain()
        plsc.subcore_barrier()
        sweep(u2_hbm)
        plsc.subcore_barrier()
        dump(o2_hbm)

    return segsum2


def _segsum_pair_call(u1, u2, src, dst, sel):
    return _make_segsum_pair()(u1, u2, src, dst, sel)


def kernel(x, hs_init, params, edge_index, gate, forward_level):
    p = params
    src = edge_index[0]
    dst = edge_index[1]
    hs = hs_init
    hf = jnp.broadcast_to(p['We'] + p['be'], (N, H))
    node_state = jnp.concatenate([hs, hf], axis=-1)
    and_m = gate == 1
    not_m = gate == 2

    masks = []
    for level in (1, 2, 3):
        lm = forward_level == level
        sa_b = lm & and_m
        sn_b = lm & not_m
        masks.append((sa_b.astype(_f32).reshape(N, 1), _pack_bits(sa_b),
                      sn_b.astype(_f32).reshape(N, 1), _pack_bits(sn_b)))

    u_af, u_as = _mlp_pair(node_state, hs, p, 'af', 'as')

    prob = None
    for level, (sa, sa_i, sn, sn_i) in zip((1, 2, 3), masks):
        ms, mf = _segsum_pair_call(u_as, u_af, src, dst, sa_i)
        hs, hf, u_ns, u_nf = _gru_mlp(ms, mf, hs, hf, sa, p,
                                      'gas', 'gaf', 'ns', 'nf', 'f')
        ms, mf = _segsum_pair_call(u_ns, u_nf, src, dst, sn_i)
        if level < 3:
            hs, hf, u_as, _ = _gru_mlp(ms, mf, hs, hf, sn, p,
                                       'gns', 'gnf', 'as', 'as', 'dup')
        else:
            hs, hf, prob = _gru_readout(ms, mf, hs, hf, sn, p, 'gns', 'gnf')

    return hs, hf, prob, jnp.float32(0.0)


# speedup vs baseline: 10.7008x; 1.0188x over previous
import functools

import jax
import jax.numpy as jnp
from jax import lax
from jax.experimental import pallas as pl
from jax.experimental.pallas import tpu as pltpu
from jax.experimental.pallas import tpu_sc as plsc

N = 10000
E = 160000
H = 128
MM = 128
NB = 2000
CHUNK = 128
NCHUNK = E // CHUNK
SELW = 512

_f32 = jnp.float32


def _pack_bits(sel_bool):
    bits = jnp.zeros((SELW * 32,), jnp.uint32).at[:N].set(
        sel_bool.astype(jnp.uint32))
    words = (bits.reshape(SELW, 32)
             << jnp.arange(32, dtype=jnp.uint32)[None, :]).sum(
        axis=1, dtype=jnp.uint32)
    return jax.lax.bitcast_convert_type(words, jnp.int32)


def _dot(a, b):
    return jnp.dot(a, b, preferred_element_type=_f32)


def _mlp_chain(f, w1, b1, w2, b2, w3, b3):
    h = jnp.maximum(_dot(f, w1) + b1, 0.0)
    h = jnp.maximum(_dot(h, w2) + b2, 0.0)
    return _dot(h, w3) + b3


def _mlp_one_body(f_ref, w1, b1, w2, b2, w3, b3, o_ref):
    o_ref[...] = _mlp_chain(f_ref[...], w1[...], b1[...], w2[...], b2[...],
                            w3[...], b3[...])


def _mlp_pair_body(fa_ref, fb_ref,
                   aw1, ab1, aw2, ab2, aw3, ab3,
                   bw1, bb1, bw2, bb2, bw3, bb3,
                   oa_ref, ob_ref):
    oa_ref[...] = _mlp_chain(fa_ref[...], aw1[...], ab1[...], aw2[...],
                             ab2[...], aw3[...], ab3[...])
    ob_ref[...] = _mlp_chain(fb_ref[...], bw1[...], bb1[...], bw2[...],
                             bb2[...], bw3[...], bb3[...])


def _row_spec(din):
    return pl.BlockSpec((NB, din), lambda i: (i, 0))


def _full_spec(shape):
    return pl.BlockSpec(shape, lambda i: tuple(0 for _ in shape))


def _mlp_weights(p, name):
    return (p[name + '_W1'], p[name + '_b1'].reshape(1, MM),
            p[name + '_W2'], p[name + '_b2'].reshape(1, MM),
            p[name + '_W3'], p[name + '_b3'].reshape(1, H))


def _mlp_one(feat, p, name):
    din = feat.shape[1]
    ws = _mlp_weights(p, name)
    wspecs = [_full_spec(w.shape) for w in ws]
    return pl.pallas_call(
        _mlp_one_body,
        grid=(N // NB,),
        in_specs=[_row_spec(din)] + wspecs,
        out_specs=_row_spec(H),
        out_shape=jax.ShapeDtypeStruct((N, H), _f32),
    )(feat, *ws)


def _mlp_pair(fa, fb, p, na, nb_):
    wsa = _mlp_weights(p, na)
    wsb = _mlp_weights(p, nb_)
    wspecs = [_full_spec(w.shape) for w in wsa + wsb]
    return pl.pallas_call(
        _mlp_pair_body,
        grid=(N // NB,),
        in_specs=[_row_spec(fa.shape[1]), _row_spec(fb.shape[1])] + wspecs,
        out_specs=[_row_spec(H), _row_spec(H)],
        out_shape=[jax.ShapeDtypeStruct((N, H), _f32)] * 2,
    )(fa, fb, *wsa, *wsb)


def _gru_math(msg, h, wih_t, bih, whh_t, bhh):
    gi = _dot(msg, wih_t) + bih
    gh = _dot(h, whh_t) + bhh
    r = jax.nn.sigmoid(gi[:, :H] + gh[:, :H])
    z = jax.nn.sigmoid(gi[:, H:2 * H] + gh[:, H:2 * H])
    n = jnp.tanh(gi[:, 2 * H:] + r * gh[:, 2 * H:])
    return (1.0 - z) * n + z * h


def _gru_pair_math(ms0, ms1, mf0, mf1, hs, hf, m_ref,
                   wih_s, bih_s, whh_s, bhh_s, wih_f, bih_f, whh_f, bhh_f):
    sel = m_ref[...] > 0.5
    new_s = _gru_math(ms0 + ms1, hs, wih_s[...], bih_s[...],
                      whh_s[...], bhh_s[...])
    new_f = _gru_math(mf0 + mf1, hf, wih_f[...], bih_f[...],
                      whh_f[...], bhh_f[...])
    return jnp.where(sel, new_s, hs), jnp.where(sel, new_f, hf)


def _gru_weights(p, name):
    return (p[name + '_Wih'].T, p[name + '_bih'].reshape(1, 3 * H),
            p[name + '_Whh'].T, p[name + '_bhh'].reshape(1, 3 * H))


def _gru_mlp_body(ms0, ms1, mf0, mf1, hs_ref, hf_ref, m_ref,
                  gw1, gw2, gw3, gw4, gw5, gw6, gw7, gw8,
                  aw1, ab1, aw2, ab2, aw3, ab3,
                  bw1, bb1, bw2, bb2, bw3, bb3,
                  hs_o, hf_o, ua_o, ub_o, *, b_input):
    hs_n, hf_n = _gru_pair_math(
        ms0[...], ms1[...], mf0[...], mf1[...], hs_ref[...], hf_ref[...],
        m_ref, gw1, gw2, gw3, gw4, gw5, gw6, gw7, gw8)
    hs_o[...] = hs_n
    hf_o[...] = hf_n
    ua_o[...] = _mlp_chain(hs_n, aw1[...], ab1[...], aw2[...], ab2[...],
                           aw3[...], ab3[...])
    if b_input == 'dup':
        ub_o[...] = ua_o[...]
    else:
        ub_o[...] = _mlp_chain(hf_n, bw1[...], bb1[...], bw2[...], bb2[...],
                               bw3[...], bb3[...])


def _gru_mlp(ms, mf, hs, hf, mask, p, gs_name, gf_name, ma_name, mb_name,
             b_input):
    gws = _gru_weights(p, gs_name) + _gru_weights(p, gf_name)
    wsa = _mlp_weights(p, ma_name)
    wsb = _mlp_weights(p, mb_name)
    ws = gws + wsa + wsb
    wspecs = [_full_spec(w.shape) for w in ws]
    nblk = N // NB
    lo = pl.BlockSpec((NB, H), lambda i: (i, 0))
    hi = pl.BlockSpec((NB, H), lambda i: (i + nblk, 0))
    body = functools.partial(_gru_mlp_body, b_input=b_input)
    return pl.pallas_call(
        body,
        grid=(nblk,),
        in_specs=[lo, hi, lo, hi, _row_spec(H), _row_spec(H),
                  pl.BlockSpec((NB, 1), lambda i: (i, 0))] + wspecs,
        out_specs=[_row_spec(H)] * 4,
        out_shape=[jax.ShapeDtypeStruct((N, H), _f32)] * 4,
    )(ms, ms, mf, mf, hs, hf, mask, *ws)


def _gru_readout_body(ms0, ms1, mf0, mf1, hs_ref, hf_ref, m_ref,
                      gw1, gw2, gw3, gw4, gw5, gw6, gw7, gw8,
                      rw1, rb1, rw2, rb2, rw3, rb3,
                      hs_o, hf_o, pr_o):
    hs_n, hf_n = _gru_pair_math(
        ms0[...], ms1[...], mf0[...], mf1[...], hs_ref[...], hf_ref[...],
        m_ref, gw1, gw2, gw3, gw4, gw5, gw6, gw7, gw8)
    hs_o[...] = hs_n
    hf_o[...] = hf_n
    h = jnp.maximum(_dot(hf_n, rw1[...]) + rb1[...], 0.0)
    h = jnp.maximum(_dot(h, rw2[...]) + rb2[...], 0.0)
    pr_o[...] = _dot(h, rw3[...]) + rb3[...]


def _readout_weights(p):
    scale = 1.0 / jnp.sqrt(jnp.float32(1.0 + 1e-5))
    w1 = p['Wp1'] * (scale * p['g1'])[None, :]
    b1 = (p['bp1'] * scale * p['g1'] + p['be1']).reshape(1, MM)
    w2 = p['Wp2'] * (scale * p['g2'])[None, :]
    b2 = (p['bp2'] * scale * p['g2'] + p['be2']).reshape(1, MM)
    return (w1, b1, w2, b2, p['Wp3'], p['bp3'].reshape(1, 1))


def _gru_readout(ms, mf, hs, hf, mask, p, gs_name, gf_name):
    ws = _gru_weights(p, gs_name) + _gru_weights(p, gf_name)
    ws = ws + _readout_weights(p)
    wspecs = [_full_spec(w.shape) for w in ws]
    nblk = N // NB
    lo = pl.BlockSpec((NB, H), lambda i: (i, 0))
    hi = pl.BlockSpec((NB, H), lambda i: (i + nblk, 0))
    return pl.pallas_call(
        _gru_readout_body,
        grid=(nblk,),
        in_specs=[lo, hi, lo, hi, _row_spec(H), _row_spec(H),
                  pl.BlockSpec((NB, 1), lambda i: (i, 0))] + wspecs,
        out_specs=[_row_spec(H), _row_spec(H),
                   pl.BlockSpec((NB, 1), lambda i: (i, 0))],
        out_shape=[jax.ShapeDtypeStruct((N, H), _f32),
                   jax.ShapeDtypeStruct((N, H), _f32),
                   jax.ShapeDtypeStruct((N, 1), _f32)],
    )(ms, ms, mf, mf, hs, hf, mask, *ws)


@functools.cache
def _make_segsum_pair():
    info = plsc.get_sparse_core_info()
    nc, ns = info.num_cores, info.num_subcores
    nw = nc * ns
    ept = E // nw
    stg = 2048
    gch = 64
    maxch = (ept + gch - 1) // gch + 1
    rows_per_tile = (N // (8 * ns)) * 8
    tail_base = rows_per_tile * ns
    tail_rows = N - tail_base
    zcopies = (rows_per_tile + gch - 1) // gch
    mesh = plsc.VectorSubcoreMesh(core_axis_name="c", subcore_axis_name="s")

    @functools.partial(
        pl.kernel, mesh=mesh,
        out_type=[jax.ShapeDtypeStruct((nc * N, H), _f32)] * 2,
        compiler_params=pltpu.CompilerParams(needs_layout_passes=False),
        scratch_types=[
            pltpu.VMEM((stg,), jnp.int32),
            pltpu.VMEM((stg,), jnp.int32),
            pltpu.VMEM((SELW,), jnp.int32),
            pltpu.VMEM((2 * gch,), jnp.int32),
            pltpu.VMEM((2 * gch,), jnp.int32),
            pltpu.VMEM((maxch, gch), jnp.int32),
            pltpu.VMEM((maxch, gch), jnp.int32),
            pltpu.VMEM((gch, H), _f32),
            pltpu.VMEM((gch, H), _f32),
            pltpu.VMEM_SHARED((N + 8, H), _f32),
            pltpu.SemaphoreType.DMA,
            pltpu.SemaphoreType.DMA,
        ])
    def segsum2(u1_hbm, u2_hbm, src_hbm, dst_hbm, sel_hbm, o1_hbm, o2_hbm,
                es, ed, sel_v, pend_s, pend_d, cs2, cd2,
                rows_a, rows_b, acc, sem_a, sem_b):
        cid = lax.axis_index("c")
        sid = lax.axis_index("s")
        wid = sid * nc + cid
        tbase = sid * rows_per_tile

        pltpu.sync_copy(sel_hbm, sel_v)

        def zero_rows_a():
            def zrow(j, _):
                rows_a[j // (H // 16), pl.ds((j % (H // 16)) * 16, 16)] = (
                    jnp.zeros((16,), _f32))
                return 0
            lax.fori_loop(0, gch * (H // 16), zrow, 0)

        def zero_acc_fire():
            for k in range(zcopies):
                nrows = min(gch, rows_per_tile - k * gch)
                pltpu.async_copy(rows_a.at[pl.ds(0, nrows)],
                                 acc.at[pl.ds(tbase + k * gch, nrows)], sem_a)

            @pl.when(sid == 0)
            def _():
                pltpu.async_copy(rows_a.at[pl.ds(0, tail_rows)],
                                 acc.at[pl.ds(tail_base, tail_rows)], sem_a)

        def zero_acc_drain():
            for k in range(zcopies):
                nrows = min(gch, rows_per_tile - k * gch)
                pltpu.make_async_copy(
                    rows_a.at[pl.ds(0, nrows)],
                    acc.at[pl.ds(tbase + k * gch, nrows)], sem_a).wait()

            @pl.when(sid == 0)
            def _():
                pltpu.make_async_copy(
                    rows_a.at[pl.ds(0, tail_rows)],
                    acc.at[pl.ds(tail_base, tail_rows)], sem_a).wait()

        def flush(carry):
            pcnt, crow = carry
            for k in range(gch // 16):
                sl = pl.ds(k * 16, 16)
                cs2[crow, sl] = pend_s[sl]
                cd2[crow, sl] = pend_d[sl]
                pend_s[sl] = pend_s[pl.ds(gch + k * 16, 16)]
                pend_d[sl] = pend_d[pl.ds(gch + k * 16, 16)]
            return pcnt - gch, crow + 1

        lane = lax.iota(jnp.int32, 16)

        def make_group_body(half_len):
            def group_body(g, carry):
                pcnt, crow = carry
                srcv = es[pl.ds(g * 16, 16)]
                dstv = ed[pl.ds(g * 16, 16)]
                w = plsc.load_gather(sel_v, [lax.shift_right_logical(dstv, 5)])
                m = (lax.shift_right_logical(w, dstv & 31) & 1) > 0
                if half_len % 16 != 0:
                    m = m & (g * 16 + lane < half_len)
                plsc.store_compressed(pend_s.at[pl.ds(pcnt, 16)], srcv,
                                      mask=m)
                plsc.store_compressed(pend_d.at[pl.ds(pcnt, 16)], dstv,
                                      mask=m)
                pcnt = pcnt + jnp.sum(m.astype(jnp.int32))
                return lax.cond(pcnt >= gch, flush, lambda c: c, (pcnt, crow))
            return group_body

        zero_rows_a()
        zero_acc_fire()

        carry = (jnp.int32(0), jnp.int32(0))
        hbase = 0
        while hbase < ept:
            hlen = min(stg, ept - hbase)
            pltpu.sync_copy(src_hbm.at[pl.ds(wid * ept + hbase, hlen)],
                            es.at[pl.ds(0, hlen)])
            pltpu.sync_copy(dst_hbm.at[pl.ds(wid * ept + hbase, hlen)],
                            ed.at[pl.ds(0, hlen)])
            carry = lax.fori_loop(0, (hlen + 15) // 16,
                                  make_group_body(hlen), carry)
            hbase += hlen
        pcnt, crow = carry
        dummy_s = jnp.zeros((16,), jnp.int32)
        dummy_d = jnp.full((16,), N, jnp.int32)
        for k in range(gch // 16):
            pend_s[pl.ds(pcnt + k * 16, 16)] = dummy_s
            pend_d[pl.ds(pcnt + k * 16, 16)] = dummy_d
        _, crow = flush((pcnt, crow))
        nch = crow

        def sweep(u_hbm):
            @pl.when(nch > 0)
            def _():
                pltpu.async_copy(u_hbm.at[cs2.at[0]], rows_a, sem_a)

            def pair_body(j2, _):
                c0 = j2 * 2
                c1 = c0 + 1

                @pl.when(c0 < nch)
                def _():
                    pltpu.make_async_copy(
                        u_hbm.at[cs2.at[c0]], rows_a, sem_a).wait()

                    @pl.when(c1 < nch)
                    def _():
                        pltpu.async_copy(u_hbm.at[cs2.at[c1]], rows_b, sem_b)
                    pltpu.sync_copy(rows_a, acc.at[cd2.at[c0]], add=True)

                @pl.when(c1 < nch)
                def _():
                    pltpu.make_async_copy(
                        u_hbm.at[cs2.at[c1]], rows_b, sem_b).wait()

                    @pl.when(c0 + 2 < nch)
                    def _():
                        pltpu.async_copy(u_hbm.at[cs2.at[c0 + 2]], rows_a,
                                         sem_a)
                    pltpu.sync_copy(rows_b, acc.at[cd2.at[c1]], add=True)
                return 0
            lax.fori_loop(0, (nch + 1) // 2, pair_body, 0)

        def dump(o_hbm):
            pltpu.sync_copy(acc.at[pl.ds(tbase, rows_per_tile)],
                            o_hbm.at[pl.ds(cid * N + tbase, rows_per_tile)])

            @pl.when(sid == 0)
            def _():
                pltpu.sync_copy(acc.at[pl.ds(tail_base, tail_rows)],
                                o_hbm.at[pl.ds(cid * N + tail_base,
                                               tail_rows)])

        zero_acc_drain()
        plsc.subcore_barrier()
        sweep(u1_hbm)
        plsc.subcore_barrier()
        dump(o1_hbm)
        zero_rows_a()
        zero_acc_fire()
        zero_acc_drain()
        plsc.subcore_barrier()
        sweep(u2_hbm)
        plsc.subcore_barrier()
        dump(o2_hbm)

    return segsum2


def _segsum_pair_call(u1, u2, src, dst, sel):
    return _make_segsum_pair()(u1, u2, src, dst, sel)


def kernel(x, hs_init, params, edge_index, gate, forward_level):
    p = params
    src = edge_index[0]
    dst = edge_index[1]
    hs = hs_init
    hf = jnp.broadcast_to(p['We'] + p['be'], (N, H))
    node_state = jnp.concatenate([hs, hf], axis=-1)
    and_m = gate == 1
    not_m = gate == 2

    masks = []
    for level in (1, 2, 3):
        lm = forward_level == level
        sa_b = lm & and_m
        sn_b = lm & not_m
        masks.append((sa_b.astype(_f32).reshape(N, 1), _pack_bits(sa_b),
                      sn_b.astype(_f32).reshape(N, 1), _pack_bits(sn_b)))

    u_af, u_as = _mlp_pair(node_state, hs, p, 'af', 'as')

    prob = None
    for level, (sa, sa_i, sn, sn_i) in zip((1, 2, 3), masks):
        ms, mf = _segsum_pair_call(u_as, u_af, src, dst, sa_i)
        hs, hf, u_ns, u_nf = _gru_mlp(ms, mf, hs, hf, sa, p,
                                      'gas', 'gaf', 'ns', 'nf', 'f')
        ms, mf = _segsum_pair_call(u_ns, u_nf, src, dst, sn_i)
        if level < 3:
            hs, hf, u_as, _ = _gru_mlp(ms, mf, hs, hf, sn, p,
                                       'gns', 'gnf', 'as', 'as', 'dup')
        else:
            hs, hf, prob = _gru_readout(ms, mf, hs, hf, sn, p, 'gns', 'gnf')

    return hs, hf, prob, jnp.float32(0.0)


# speedup vs baseline: 10.8971x; 1.0183x over previous
import functools

import jax
import jax.numpy as jnp
from jax import lax
from jax.experimental import pallas as pl
from jax.experimental.pallas import tpu as pltpu
from jax.experimental.pallas import tpu_sc as plsc

N = 10000
E = 160000
H = 128
MM = 128
NB = 2000
CHUNK = 128
NCHUNK = E // CHUNK
SELW = 512

_f32 = jnp.float32


def _pack_bits(sel_bool):
    bits = jnp.zeros((SELW * 32,), jnp.uint32).at[:N].set(
        sel_bool.astype(jnp.uint32))
    words = (bits.reshape(SELW, 32)
             << jnp.arange(32, dtype=jnp.uint32)[None, :]).sum(
        axis=1, dtype=jnp.uint32)
    return jax.lax.bitcast_convert_type(words, jnp.int32)


def _dot(a, b):
    return jnp.dot(a, b, preferred_element_type=_f32)


def _mlp_chain(f, w1, b1, w2, b2, w3, b3):
    h = jnp.maximum(_dot(f, w1) + b1, 0.0)
    h = jnp.maximum(_dot(h, w2) + b2, 0.0)
    return _dot(h, w3) + b3


def _mlp_one_body(f_ref, w1, b1, w2, b2, w3, b3, o_ref):
    o_ref[...] = _mlp_chain(f_ref[...], w1[...], b1[...], w2[...], b2[...],
                            w3[...], b3[...])


def _mlp_pair_body(fa_ref, fb_ref,
                   aw1, ab1, aw2, ab2, aw3, ab3,
                   bw1, bb1, bw2, bb2, bw3, bb3,
                   oa_ref, ob_ref):
    oa_ref[...] = _mlp_chain(fa_ref[...], aw1[...], ab1[...], aw2[...],
                             ab2[...], aw3[...], ab3[...])
    ob_ref[...] = _mlp_chain(fb_ref[...], bw1[...], bb1[...], bw2[...],
                             bb2[...], bw3[...], bb3[...])


def _row_spec(din):
    return pl.BlockSpec((NB, din), lambda i: (i, 0))


def _full_spec(shape):
    return pl.BlockSpec(shape, lambda i: tuple(0 for _ in shape))


def _mlp_weights(p, name):
    return (p[name + '_W1'], p[name + '_b1'].reshape(1, MM),
            p[name + '_W2'], p[name + '_b2'].reshape(1, MM),
            p[name + '_W3'], p[name + '_b3'].reshape(1, H))


def _mlp_one(feat, p, name):
    din = feat.shape[1]
    ws = _mlp_weights(p, name)
    wspecs = [_full_spec(w.shape) for w in ws]
    return pl.pallas_call(
        _mlp_one_body,
        grid=(N // NB,),
        in_specs=[_row_spec(din)] + wspecs,
        out_specs=_row_spec(H),
        out_shape=jax.ShapeDtypeStruct((N, H), _f32),
    )(feat, *ws)


def _mlp_pair(fa, fb, p, na, nb_):
    wsa = _mlp_weights(p, na)
    wsb = _mlp_weights(p, nb_)
    wspecs = [_full_spec(w.shape) for w in wsa + wsb]
    return pl.pallas_call(
        _mlp_pair_body,
        grid=(N // NB,),
        in_specs=[_row_spec(fa.shape[1]), _row_spec(fb.shape[1])] + wspecs,
        out_specs=[_row_spec(H), _row_spec(H)],
        out_shape=[jax.ShapeDtypeStruct((N, H), _f32)] * 2,
    )(fa, fb, *wsa, *wsb)


def _gru_math(msg, h, wih_t, bih, whh_t, bhh):
    gi = _dot(msg, wih_t) + bih
    gh = _dot(h, whh_t) + bhh
    r = jax.nn.sigmoid(gi[:, :H] + gh[:, :H])
    z = jax.nn.sigmoid(gi[:, H:2 * H] + gh[:, H:2 * H])
    n = jnp.tanh(gi[:, 2 * H:] + r * gh[:, 2 * H:])
    return (1.0 - z) * n + z * h


def _gru_pair_math(ms0, ms1, mf0, mf1, hs, hf, m_ref,
                   wih_s, bih_s, whh_s, bhh_s, wih_f, bih_f, whh_f, bhh_f):
    sel = m_ref[...] > 0.5
    new_s = _gru_math(ms0 + ms1, hs, wih_s[...], bih_s[...],
                      whh_s[...], bhh_s[...])
    new_f = _gru_math(mf0 + mf1, hf, wih_f[...], bih_f[...],
                      whh_f[...], bhh_f[...])
    return jnp.where(sel, new_s, hs), jnp.where(sel, new_f, hf)


def _gru_weights(p, name):
    return (p[name + '_Wih'].T, p[name + '_bih'].reshape(1, 3 * H),
            p[name + '_Whh'].T, p[name + '_bhh'].reshape(1, 3 * H))


def _gru_mlp_body(ms0, ms1, mf0, mf1, hs_ref, hf_ref, m_ref,
                  gw1, gw2, gw3, gw4, gw5, gw6, gw7, gw8,
                  aw1, ab1, aw2, ab2, aw3, ab3,
                  bw1, bb1, bw2, bb2, bw3, bb3,
                  hs_o, hf_o, ua_o, ub_o, *, b_input):
    hs_n, hf_n = _gru_pair_math(
        ms0[...], ms1[...], mf0[...], mf1[...], hs_ref[...], hf_ref[...],
        m_ref, gw1, gw2, gw3, gw4, gw5, gw6, gw7, gw8)
    hs_o[...] = hs_n
    hf_o[...] = hf_n
    ua_o[...] = _mlp_chain(hs_n, aw1[...], ab1[...], aw2[...], ab2[...],
                           aw3[...], ab3[...])
    if b_input == 'dup':
        ub_o[...] = ua_o[...]
    else:
        ub_o[...] = _mlp_chain(hf_n, bw1[...], bb1[...], bw2[...], bb2[...],
                               bw3[...], bb3[...])


def _gru_mlp(ms, mf, hs, hf, mask, p, gs_name, gf_name, ma_name, mb_name,
             b_input):
    gws = _gru_weights(p, gs_name) + _gru_weights(p, gf_name)
    wsa = _mlp_weights(p, ma_name)
    wsb = _mlp_weights(p, mb_name)
    ws = gws + wsa + wsb
    wspecs = [_full_spec(w.shape) for w in ws]
    nblk = N // NB
    lo = pl.BlockSpec((NB, H), lambda i: (i, 0))
    hi = pl.BlockSpec((NB, H), lambda i: (i + nblk, 0))
    body = functools.partial(_gru_mlp_body, b_input=b_input)
    return pl.pallas_call(
        body,
        grid=(nblk,),
        in_specs=[lo, hi, lo, hi, _row_spec(H), _row_spec(H),
                  pl.BlockSpec((NB, 1), lambda i: (i, 0))] + wspecs,
        out_specs=[_row_spec(H)] * 4,
        out_shape=[jax.ShapeDtypeStruct((N, H), _f32)] * 4,
    )(ms, ms, mf, mf, hs, hf, mask, *ws)


def _gru_readout_body(ms0, ms1, mf0, mf1, hs_ref, hf_ref, m_ref,
                      gw1, gw2, gw3, gw4, gw5, gw6, gw7, gw8,
                      rw1, rb1, rw2, rb2, rw3, rb3,
                      hs_o, hf_o, pr_o):
    hs_n, hf_n = _gru_pair_math(
        ms0[...], ms1[...], mf0[...], mf1[...], hs_ref[...], hf_ref[...],
        m_ref, gw1, gw2, gw3, gw4, gw5, gw6, gw7, gw8)
    hs_o[...] = hs_n
    hf_o[...] = hf_n
    h = jnp.maximum(_dot(hf_n, rw1[...]) + rb1[...], 0.0)
    h = jnp.maximum(_dot(h, rw2[...]) + rb2[...], 0.0)
    pr_o[...] = _dot(h, rw3[...]) + rb3[...]


def _readout_weights(p):
    scale = 1.0 / jnp.sqrt(jnp.float32(1.0 + 1e-5))
    w1 = p['Wp1'] * (scale * p['g1'])[None, :]
    b1 = (p['bp1'] * scale * p['g1'] + p['be1']).reshape(1, MM)
    w2 = p['Wp2'] * (scale * p['g2'])[None, :]
    b2 = (p['bp2'] * scale * p['g2'] + p['be2']).reshape(1, MM)
    return (w1, b1, w2, b2, p['Wp3'], p['bp3'].reshape(1, 1))


def _gru_readout(ms, mf, hs, hf, mask, p, gs_name, gf_name):
    ws = _gru_weights(p, gs_name) + _gru_weights(p, gf_name)
    ws = ws + _readout_weights(p)
    wspecs = [_full_spec(w.shape) for w in ws]
    nblk = N // NB
    lo = pl.BlockSpec((NB, H), lambda i: (i, 0))
    hi = pl.BlockSpec((NB, H), lambda i: (i + nblk, 0))
    return pl.pallas_call(
        _gru_readout_body,
        grid=(nblk,),
        in_specs=[lo, hi, lo, hi, _row_spec(H), _row_spec(H),
                  pl.BlockSpec((NB, 1), lambda i: (i, 0))] + wspecs,
        out_specs=[_row_spec(H), _row_spec(H),
                   pl.BlockSpec((NB, 1), lambda i: (i, 0))],
        out_shape=[jax.ShapeDtypeStruct((N, H), _f32),
                   jax.ShapeDtypeStruct((N, H), _f32),
                   jax.ShapeDtypeStruct((N, 1), _f32)],
    )(ms, ms, mf, mf, hs, hf, mask, *ws)


@functools.cache
def _make_segsum_pair():
    info = plsc.get_sparse_core_info()
    nc, ns = info.num_cores, info.num_subcores
    nw = nc * ns
    ept = E // nw
    stg = 2048
    gch = 64
    maxch = (ept + gch - 1) // gch + 1
    rows_per_tile = (N // (8 * ns)) * 8
    tail_base = rows_per_tile * ns
    tail_rows = N - tail_base
    zcopies = (rows_per_tile + gch - 1) // gch
    mesh = plsc.VectorSubcoreMesh(core_axis_name="c", subcore_axis_name="s")

    @functools.partial(
        pl.kernel, mesh=mesh,
        out_type=[jax.ShapeDtypeStruct((nc * N, H), _f32)] * 2,
        compiler_params=pltpu.CompilerParams(needs_layout_passes=False),
        scratch_types=[
            pltpu.VMEM((stg,), jnp.int32),
            pltpu.VMEM((stg,), jnp.int32),
            pltpu.VMEM((stg,), jnp.int32),
            pltpu.VMEM((stg,), jnp.int32),
            pltpu.VMEM((SELW,), jnp.int32),
            pltpu.VMEM((2 * gch,), jnp.int32),
            pltpu.VMEM((2 * gch,), jnp.int32),
            pltpu.VMEM((maxch, gch), jnp.int32),
            pltpu.VMEM((maxch, gch), jnp.int32),
            pltpu.VMEM((gch, H), _f32),
            pltpu.VMEM((gch, H), _f32),
            pltpu.VMEM_SHARED((N + 8, H), _f32),
            pltpu.SemaphoreType.DMA,
            pltpu.SemaphoreType.DMA,
            pltpu.SemaphoreType.DMA,
        ])
    def segsum2(u1_hbm, u2_hbm, src_hbm, dst_hbm, sel_hbm, o1_hbm, o2_hbm,
                es0, ed0, es1, ed1, sel_v, pend_s, pend_d, cs2, cd2,
                rows_a, rows_b, acc, sem_a, sem_b, sem_c):
        cid = lax.axis_index("c")
        sid = lax.axis_index("s")
        wid = sid * nc + cid
        tbase = sid * rows_per_tile

        pltpu.sync_copy(sel_hbm, sel_v)

        def zero_rows_a():
            def zrow(j, _):
                rows_a[j // (H // 16), pl.ds((j % (H // 16)) * 16, 16)] = (
                    jnp.zeros((16,), _f32))
                return 0
            lax.fori_loop(0, gch * (H // 16), zrow, 0)

        def zero_acc_fire():
            for k in range(zcopies):
                nrows = min(gch, rows_per_tile - k * gch)
                pltpu.async_copy(rows_a.at[pl.ds(0, nrows)],
                                 acc.at[pl.ds(tbase + k * gch, nrows)], sem_a)

            @pl.when(sid == 0)
            def _():
                pltpu.async_copy(rows_a.at[pl.ds(0, tail_rows)],
                                 acc.at[pl.ds(tail_base, tail_rows)], sem_a)

        def zero_acc_drain():
            for k in range(zcopies):
                nrows = min(gch, rows_per_tile - k * gch)
                pltpu.make_async_copy(
                    rows_a.at[pl.ds(0, nrows)],
                    acc.at[pl.ds(tbase + k * gch, nrows)], sem_a).wait()

            @pl.when(sid == 0)
            def _():
                pltpu.make_async_copy(
                    rows_a.at[pl.ds(0, tail_rows)],
                    acc.at[pl.ds(tail_base, tail_rows)], sem_a).wait()

        def flush(carry):
            pcnt, crow = carry
            for k in range(gch // 16):
                sl = pl.ds(k * 16, 16)
                cs2[crow, sl] = pend_s[sl]
                cd2[crow, sl] = pend_d[sl]
                pend_s[sl] = pend_s[pl.ds(gch + k * 16, 16)]
                pend_d[sl] = pend_d[pl.ds(gch + k * 16, 16)]
            return pcnt - gch, crow + 1

        lane = lax.iota(jnp.int32, 16)

        ebufs = ((es0, ed0), (es1, ed1))

        def make_group_body(half_len, hb):
            es, ed = ebufs[hb]

            def group_body(g, carry):
                pcnt, crow = carry
                srcv = es[pl.ds(g * 16, 16)]
                dstv = ed[pl.ds(g * 16, 16)]
                w = plsc.load_gather(sel_v, [lax.shift_right_logical(dstv, 5)])
                m = (lax.shift_right_logical(w, dstv & 31) & 1) > 0
                if half_len % 16 != 0:
                    m = m & (g * 16 + lane < half_len)
                plsc.store_compressed(pend_s.at[pl.ds(pcnt, 16)], srcv,
                                      mask=m)
                plsc.store_compressed(pend_d.at[pl.ds(pcnt, 16)], dstv,
                                      mask=m)
                pcnt = pcnt + jnp.sum(m.astype(jnp.int32))
                return lax.cond(pcnt >= gch, flush, lambda c: c, (pcnt, crow))
            return group_body

        zero_rows_a()
        zero_acc_fire()

        rounds = []
        hbase = 0
        while hbase < ept:
            hlen = min(stg, ept - hbase)
            rounds.append((hbase, hlen))
            hbase += hlen

        def stage(r, hb):
            hbase, hlen = rounds[r]
            es, ed = ebufs[hb]
            pltpu.async_copy(src_hbm.at[pl.ds(wid * ept + hbase, hlen)],
                             es.at[pl.ds(0, hlen)], sem_c)
            pltpu.async_copy(dst_hbm.at[pl.ds(wid * ept + hbase, hlen)],
                             ed.at[pl.ds(0, hlen)], sem_c)

        def stage_wait(r, hb):
            hbase, hlen = rounds[r]
            es, ed = ebufs[hb]
            pltpu.make_async_copy(
                src_hbm.at[pl.ds(wid * ept + hbase, hlen)],
                es.at[pl.ds(0, hlen)], sem_c).wait()
            pltpu.make_async_copy(
                dst_hbm.at[pl.ds(wid * ept + hbase, hlen)],
                ed.at[pl.ds(0, hlen)], sem_c).wait()

        stage(0, 0)
        carry = (jnp.int32(0), jnp.int32(0))
        for r, (hbase, hlen) in enumerate(rounds):
            hb = r % 2
            stage_wait(r, hb)
            if r + 1 < len(rounds):
                stage(r + 1, (r + 1) % 2)
            carry = lax.fori_loop(0, (hlen + 15) // 16,
                                  make_group_body(hlen, hb), carry)
        pcnt, crow = carry
        dummy_s = jnp.zeros((16,), jnp.int32)
        dummy_d = jnp.full((16,), N, jnp.int32)
        for k in range(gch // 16):
            pend_s[pl.ds(pcnt + k * 16, 16)] = dummy_s
            pend_d[pl.ds(pcnt + k * 16, 16)] = dummy_d
        _, crow = flush((pcnt, crow))
        nch = crow

        def sweep(u_hbm):
            @pl.when(nch > 0)
            def _():
                pltpu.async_copy(u_hbm.at[cs2.at[0]], rows_a, sem_a)

            def pair_body(j2, _):
                c0 = j2 * 2
                c1 = c0 + 1

                @pl.when(c0 < nch)
                def _():
                    pltpu.make_async_copy(
                        u_hbm.at[cs2.at[c0]], rows_a, sem_a).wait()

                    @pl.when(c1 < nch)
                    def _():
                        pltpu.async_copy(u_hbm.at[cs2.at[c1]], rows_b, sem_b)
                    pltpu.sync_copy(rows_a, acc.at[cd2.at[c0]], add=True)

                @pl.when(c1 < nch)
                def _():
                    pltpu.make_async_copy(
                        u_hbm.at[cs2.at[c1]], rows_b, sem_b).wait()

                    @pl.when(c0 + 2 < nch)
                    def _():
                        pltpu.async_copy(u_hbm.at[cs2.at[c0 + 2]], rows_a,
                                         sem_a)
                    pltpu.sync_copy(rows_b, acc.at[cd2.at[c1]], add=True)
                return 0
            lax.fori_loop(0, (nch + 1) // 2, pair_body, 0)

        def dump(o_hbm):
            pltpu.sync_copy(acc.at[pl.ds(tbase, rows_per_tile)],
                            o_hbm.at[pl.ds(cid * N + tbase, rows_per_tile)])

            @pl.when(sid == 0)
            def _():
                pltpu.sync_copy(acc.at[pl.ds(tail_base, tail_rows)],
                                o_hbm.at[pl.ds(cid * N + tail_base,
                                               tail_rows)])

        zero_acc_drain()
        plsc.subcore_barrier()
        sweep(u1_hbm)
        plsc.subcore_barrier()
        dump(o1_hbm)
        zero_rows_a()
        zero_acc_fire()
        zero_acc_drain()
        plsc.subcore_barrier()
        sweep(u2_hbm)
        plsc.subcore_barrier()
        dump(o2_hbm)

    return segsum2


def _segsum_pair_call(u1, u2, src, dst, sel):
    return _make_segsum_pair()(u1, u2, src, dst, sel)


def kernel(x, hs_init, params, edge_index, gate, forward_level):
    p = params
    src = edge_index[0]
    dst = edge_index[1]
    hs = hs_init
    hf = jnp.broadcast_to(p['We'] + p['be'], (N, H))
    node_state = jnp.concatenate([hs, hf], axis=-1)
    and_m = gate == 1
    not_m = gate == 2

    masks = []
    for level in (1, 2, 3):
        lm = forward_level == level
        sa_b = lm & and_m
        sn_b = lm & not_m
        masks.append((sa_b.astype(_f32).reshape(N, 1), _pack_bits(sa_b),
                      sn_b.astype(_f32).reshape(N, 1), _pack_bits(sn_b)))

    u_af, u_as = _mlp_pair(node_state, hs, p, 'af', 'as')

    prob = None
    for level, (sa, sa_i, sn, sn_i) in zip((1, 2, 3), masks):
        ms, mf = _segsum_pair_call(u_as, u_af, src, dst, sa_i)
        hs, hf, u_ns, u_nf = _gru_mlp(ms, mf, hs, hf, sa, p,
                                      'gas', 'gaf', 'ns', 'nf', 'f')
        ms, mf = _segsum_pair_call(u_ns, u_nf, src, dst, sn_i)
        if level < 3:
            hs, hf, u_as, _ = _gru_mlp(ms, mf, hs, hf, sn, p,
                                       'gns', 'gnf', 'as', 'as', 'dup')
        else:
            hs, hf, prob = _gru_readout(ms, mf, hs, hf, sn, p, 'gns', 'gnf')

    return hs, hf, prob, jnp.float32(0.0)
